# baseline (speedup 1.0000x reference)
import jax
import jax.numpy as jnp
from jax import lax
from jax.experimental import pallas as pl
from jax.experimental.pallas import tpu as pltpu
from jax.experimental.pallas import tpu_sc as plsc

VOCAB = 1000000
D = 64
ROWS = 16384
COLS = 50
NC = 2
NS = 16
NW = NC * NS
ROWS_PER_W = ROWS // NW
GROUP = 8
GROUPS = ROWS_PER_W // GROUP


def _fire(table_hbm, idx_v, rows_buf, sem, g):
    for j in range(GROUP):
        pltpu.async_copy(
            table_hbm.at[idx_v.at[g * GROUP + j]],
            rows_buf.at[j],
            sem,
        )


def _drain(table_hbm, idx_v, rows_buf, sem):
    for j in range(GROUP):
        pltpu.make_async_copy(
            table_hbm.at[idx_v.at[j]],
            rows_buf.at[j],
            sem,
        ).wait()


def _emb_body(x_hbm, table_hbm, out_hbm, idx_v, rows0, rows1, sem0, sem1):
    wid = lax.axis_index("s") * NC + lax.axis_index("c")
    row_base = wid * ROWS_PER_W
    pltpu.sync_copy(x_hbm.at[pl.ds(row_base, ROWS_PER_W)], idx_v)

    def store(rows_buf, g):
        pltpu.sync_copy(rows_buf, out_hbm.at[pl.ds(row_base + g * GROUP, GROUP)])

    _fire(table_hbm, idx_v, rows0, sem0, 0)

    def pair_body(i, _):
        g = 2 * i
        _drain(table_hbm, idx_v, rows0, sem0)
        _fire(table_hbm, idx_v, rows1, sem1, g + 1)
        store(rows0, g)
        _drain(table_hbm, idx_v, rows1, sem1)
        _fire(table_hbm, idx_v, rows0, sem0, g + 2)
        store(rows1, g + 1)
        return ()

    lax.fori_loop(0, GROUPS // 2 - 1, pair_body, (), unroll=False)

    g = GROUPS - 2
    _drain(table_hbm, idx_v, rows0, sem0)
    _fire(table_hbm, idx_v, rows1, sem1, g + 1)
    store(rows0, g)
    _drain(table_hbm, idx_v, rows1, sem1)
    store(rows1, g + 1)


@jax.jit
def _emb_call(x, weight):
    mesh = plsc.VectorSubcoreMesh(core_axis_name="c", subcore_axis_name="s")
    return pl.kernel(
        _emb_body,
        out_type=jax.ShapeDtypeStruct((ROWS, COLS, D), jnp.float32),
        mesh=mesh,
        scratch_types=[
            pltpu.VMEM((ROWS_PER_W, COLS), jnp.int32),
            pltpu.VMEM((GROUP, COLS, D), jnp.float32),
            pltpu.VMEM((GROUP, COLS, D), jnp.float32),
            pltpu.SemaphoreType.DMA,
            pltpu.SemaphoreType.DMA,
        ],
        compiler_params=pltpu.CompilerParams(use_tc_tiling_on_sc=False),
    )(x, weight)


def kernel(x, weight):
    return _emb_call(x.astype(jnp.int32), weight)
